# Initial kernel scaffold; baseline (speedup 1.0000x reference)
import jax
import jax.numpy as jnp
from jax.experimental import pallas as pl


def kernel(x, edge_index, edge_attr, batch, bn_feat_g, bn_feat_b, Wn0, b_n0, We0, b_e0, bns_conv_g, bns_conv_b, Wn, b_n, We, b_e, bn_fc_g, bn_fc_b, W_lin, b_lin, bn_hid_g, bn_hid_b, W_cls, b_cls):
    raise NotImplementedError("write your pallas kernel here")


# speedup vs baseline: 4.8469x; 4.8469x over previous
import functools
import jax
import jax.numpy as jnp
from jax import lax
from jax.experimental import pallas as pl
from jax.experimental.pallas import tpu as pltpu
from jax.experimental.pallas import tpu_sc as plsc

N = 10000
E = 320000
H = 128
DE = 16
NCLS = 10
BGRAPH = 128
NLAYER = 3

NCORE = 2
NSUB = 16
NWORK = NCORE * NSUB
K = 128
CPW = 79
EW = CPW * K
EP = NWORK * EW
NPAD = 10112
ZR = NPAD // NSUB
DW = 16

F32 = jnp.float32
_TC_BIG = pltpu.CompilerParams(vmem_limit_bytes=110 * 1024 * 1024)


@functools.lru_cache(maxsize=None)
def _mesh():
    return plsc.VectorSubcoreMesh(
        core_axis_name="c", subcore_axis_name="s",
        num_cores=NCORE, num_subcores=NSUB)


@functools.lru_cache(maxsize=None)
def _build_sc_degree():
    @functools.partial(
        pl.kernel,
        out_type=jax.ShapeDtypeStruct((NWORK, NPAD), F32),
        mesh=_mesh(),
        scratch_types=[
            pltpu.VMEM((K,), jnp.int32),
            pltpu.VMEM((NPAD,), F32),
        ],
        compiler_params=pltpu.CompilerParams(needs_layout_passes=False),
    )
    def sc_degree(dst_hbm, zeros_n_hbm, out_hbm, didx, hist):
        c = lax.axis_index("c")
        s = lax.axis_index("s")
        wid = c * NSUB + s
        pltpu.sync_copy(zeros_n_hbm, hist)
        base = wid * EW
        ones16 = jnp.ones((16,), F32)

        def body(i, carry):
            off = base + i * K
            pltpu.sync_copy(dst_hbm.at[pl.ds(off, K)], didx)
            for j in range(K // 16):
                idxv = didx[pl.ds(j * 16, 16)]
                plsc.addupdate_scatter(hist, [idxv], ones16)
            return carry

        lax.fori_loop(0, CPW, body, 0)
        pltpu.sync_copy(hist, out_hbm.at[wid])

    return sc_degree


def _sc_degree(dst_p, zeros_n):
    return _build_sc_degree()(dst_p, zeros_n)


@functools.lru_cache(maxsize=None)
def _build_sc_wgather():
    @functools.partial(
        pl.kernel,
        out_type=(jax.ShapeDtypeStruct((EP,), F32),
                  jax.ShapeDtypeStruct((NWORK, NPAD), F32)),
        mesh=_mesh(),
        scratch_types=[
            pltpu.VMEM((NPAD,), F32),
            pltpu.VMEM((NPAD,), F32),
            pltpu.VMEM((K,), jnp.int32),
            pltpu.VMEM((K,), jnp.int32),
            pltpu.VMEM((K,), F32),
        ],
        compiler_params=pltpu.CompilerParams(needs_layout_passes=False),
    )
    def sc_wgather(r_hbm, src_hbm, dst_hbm, zeros_n_hbm, w_hbm, t0_hbm,
                   rtab, t0h, sidx, didx, wbuf):
        c = lax.axis_index("c")
        s = lax.axis_index("s")
        wid = c * NSUB + s
        pltpu.sync_copy(r_hbm, rtab)
        pltpu.sync_copy(zeros_n_hbm, t0h)
        base = wid * EW

        def body(i, carry):
            off = base + i * K
            pltpu.sync_copy(src_hbm.at[pl.ds(off, K)], sidx)
            pltpu.sync_copy(dst_hbm.at[pl.ds(off, K)], didx)
            for j in range(K // 16):
                idxv = sidx[pl.ds(j * 16, 16)]
                w16 = plsc.load_gather(rtab, [idxv])
                wbuf[pl.ds(j * 16, 16)] = w16
                didxv = didx[pl.ds(j * 16, 16)]
                plsc.addupdate_scatter(t0h, [didxv], w16)
            pltpu.sync_copy(wbuf, w_hbm.at[pl.ds(off, K)])
            return carry

        lax.fori_loop(0, CPW, body, 0)
        pltpu.sync_copy(t0h, t0_hbm.at[wid])

    return sc_wgather


def _sc_wgather(r1d, src_p, dst_p, zeros_n):
    return _build_sc_wgather()(r1d, src_p, dst_p, zeros_n)


@functools.lru_cache(maxsize=None)
def _build_sc_scatter_u():
    scratch = [
        pltpu.VMEM((K,), jnp.int32),
        pltpu.VMEM((K,), jnp.int32),
        pltpu.VMEM((K, H), F32),
        pltpu.VMEM_SHARED((NPAD, H), F32),
        pltpu.SemaphoreType.DMA,
    ]

    @functools.partial(
        pl.kernel,
        out_type=jax.ShapeDtypeStruct((NCORE, NPAD, H), F32),
        mesh=_mesh(),
        scratch_types=scratch,
    )
    def sc_scatter(rows_hbm, src_hbm, dst_hbm, zeros_hbm, out_hbm,
                   sidx, didx, rows, acc, sem):
        c = lax.axis_index("c")
        s = lax.axis_index("s")
        wid = c * NSUB + s
        pltpu.sync_copy(zeros_hbm.at[pl.ds(s * ZR, ZR)],
                        acc.at[pl.ds(s * ZR, ZR)])
        plsc.subcore_barrier()
        base = wid * EW

        def body(i, carry):
            off = base + i * K
            pltpu.sync_copy(dst_hbm.at[pl.ds(off, K)], didx)
            pltpu.sync_copy(src_hbm.at[pl.ds(off, K)], sidx)
            pltpu.async_copy(rows_hbm.at[sidx], rows, sem).wait()
            pltpu.sync_copy(rows, acc.at[didx], add=True)
            return carry

        lax.fori_loop(0, CPW, body, 0)
        plsc.subcore_barrier()
        pltpu.sync_copy(acc.at[pl.ds(s * ZR, ZR)],
                        out_hbm.at[c].at[pl.ds(s * ZR, ZR)])

    return sc_scatter


def _sc_scatter_u(u, src_p, dst_p, zeros_u):
    return _build_sc_scatter_u()(u, src_p, dst_p, zeros_u)


@functools.lru_cache(maxsize=None)
def _build_sc_scatter_q():
    scratch = [
        pltpu.VMEM((K,), jnp.int32),
        pltpu.VMEM((K, H), F32),
        pltpu.VMEM_SHARED((NPAD, H), F32),
    ]

    @functools.partial(
        pl.kernel,
        out_type=jax.ShapeDtypeStruct((NCORE, NPAD, H), F32),
        mesh=_mesh(),
        scratch_types=scratch,
    )
    def sc_scatter(q_hbm, dst_hbm, zeros_hbm, out_hbm, didx, rows, acc):
        c = lax.axis_index("c")
        s = lax.axis_index("s")
        wid = c * NSUB + s
        pltpu.sync_copy(zeros_hbm.at[pl.ds(s * ZR, ZR)],
                        acc.at[pl.ds(s * ZR, ZR)])
        plsc.subcore_barrier()
        base = wid * EW

        def body(i, carry):
            off = base + i * K
            pltpu.sync_copy(dst_hbm.at[pl.ds(off, K)], didx)
            pltpu.sync_copy(q_hbm.at[pl.ds(off, K)], rows)
            pltpu.sync_copy(rows, acc.at[didx], add=True)
            return carry

        lax.fori_loop(0, CPW, body, 0)
        plsc.subcore_barrier()
        pltpu.sync_copy(acc.at[pl.ds(s * ZR, ZR)],
                        out_hbm.at[c].at[pl.ds(s * ZR, ZR)])

    return sc_scatter


def _sc_scatter_q(qp, dst_p, zeros_u):
    return _build_sc_scatter_q()(qp, dst_p, zeros_u)


def _rowmask():
    ids = lax.broadcasted_iota(jnp.int32, (NPAD, 1), 0)
    return (ids < N).astype(F32)


def _bn_padded(h, g, b, n_rows):
    sm = jnp.sum(h, axis=0, keepdims=True) / n_rows
    sq = jnp.sum(h * h, axis=0, keepdims=True) / n_rows
    var = sq - sm * sm
    return (h - sm) * lax.rsqrt(var + 1e-5) * g + b


def _tc_prep_body(x_ref, g_ref, b_ref, wn0_ref, bn0_ref, degp_ref,
                  h1_ref, r_ref):
    mask = _rowmask()
    deg = lax.dot_general(degp_ref[...], jnp.ones((NWORK, 1), F32),
                          (((0,), (0,)), ((), ())),
                          preferred_element_type=F32) + 1.0
    r_ref[...] = lax.rsqrt(deg) * mask
    hb = _bn_padded(x_ref[...], g_ref[...], b_ref[...], N)
    h1 = jnp.maximum(jnp.dot(hb, wn0_ref[...], preferred_element_type=F32)
                     + bn0_ref[...], 0.0)
    h1_ref[...] = h1 * mask


def _tc_prep(x_pad, g, b, wn0, bn0, degp):
    return pl.pallas_call(
        _tc_prep_body,
        compiler_params=_TC_BIG,
        out_shape=[jax.ShapeDtypeStruct((NPAD, H), F32),
                   jax.ShapeDtypeStruct((NPAD, 1), F32)],
    )(x_pad, g, b, wn0, bn0, degp)


_ETILE = 2048


def _tc_edge_body(ea_ref, we0_ref, be0_ref, w_ref, out_ref):
    z = jnp.dot(ea_ref[...], we0_ref[...], preferred_element_type=F32) + be0_ref[...]
    out_ref[...] = jnp.maximum(z, 0.0) * w_ref[...]


def _tc_edge(ea_pad, we0, be0, w2d):
    grid = (EP // _ETILE,)
    return pl.pallas_call(
        _tc_edge_body,
        grid=grid,
        in_specs=[
            pl.BlockSpec((_ETILE, DE), lambda i: (i, 0)),
            pl.BlockSpec((DE, H), lambda i: (0, 0)),
            pl.BlockSpec((1, H), lambda i: (0, 0)),
            pl.BlockSpec((_ETILE, 1), lambda i: (i, 0)),
        ],
        out_specs=pl.BlockSpec((_ETILE, H), lambda i: (i, 0)),
        out_shape=jax.ShapeDtypeStruct((EP, H), F32),
    )(ea_pad, we0, be0, w2d)


def _tc_c1_body(h1_ref, r_ref, g_ref, b_ref, wn_ref, bn_ref,
                hp_ref, u_ref):
    mask = _rowmask()
    hb = _bn_padded(h1_ref[...], g_ref[...], b_ref[...], N)
    hp = (jnp.dot(hb, wn_ref[...], preferred_element_type=F32)
          + bn_ref[...]) * mask
    hp_ref[...] = hp
    u_ref[...] = hp * r_ref[...]


def _tc_c1(h1, r, g, b, wn, bn):
    return pl.pallas_call(
        _tc_c1_body,
        compiler_params=_TC_BIG,
        out_shape=[jax.ShapeDtypeStruct((NPAD, H), F32),
                   jax.ShapeDtypeStruct((NPAD, H), F32)],
    )(h1, r, g, b, wn, bn)


def _tc_mid_body(gu_ref, sq_ref, t0_ref, hp_ref, r_ref, we_ref, be_ref,
                 g_ref, b_ref, wn_ref, bn_ref, hp2_ref, u2_ref):
    mask = _rowmask()
    r = r_ref[...]
    sqs = sq_ref[0] + sq_ref[1]
    t0 = lax.dot_general(t0_ref[...], jnp.ones((NWORK, 1), F32),
                         (((0,), (0,)), ((), ())),
                         preferred_element_type=F32)
    se = (jnp.dot(sqs, we_ref[...], preferred_element_type=F32)
          + t0 * be_ref[...])
    gu = gu_ref[0] + gu_ref[1]
    h = jnp.maximum(r * (gu + se) + hp_ref[...], 0.0)
    hb = _bn_padded(h, g_ref[...], b_ref[...], N)
    hp2 = (jnp.dot(hb, wn_ref[...], preferred_element_type=F32)
           + bn_ref[...]) * mask
    hp2_ref[...] = hp2
    u2_ref[...] = hp2 * r


def _tc_mid(gu, sqp, t0p, hp, r, we, be, g, b, wn, bn):
    return pl.pallas_call(
        _tc_mid_body,
        compiler_params=_TC_BIG,
        out_shape=[jax.ShapeDtypeStruct((NPAD, H), F32),
                   jax.ShapeDtypeStruct((NPAD, H), F32)],
    )(gu, sqp, t0p, hp, r, we, be, g, b, wn, bn)


def _tc_final_body(gu_ref, sq_ref, t0_ref, hp_ref, r_ref, we_ref, be_ref,
                   batch_ref, fg_ref, fb_ref, wl_ref, bl_ref,
                   hg_ref, hb_ref, wc_ref, bc_ref, out_ref):
    r = r_ref[...]
    sqs = sq_ref[0] + sq_ref[1]
    t0 = lax.dot_general(t0_ref[...], jnp.ones((NWORK, 1), F32),
                         (((0,), (0,)), ((), ())),
                         preferred_element_type=F32)
    se = (jnp.dot(sqs, we_ref[...], preferred_element_type=F32)
          + t0 * be_ref[...])
    gu = gu_ref[0] + gu_ref[1]
    h = jnp.maximum(r * (gu + se) + hp_ref[...], 0.0)
    cols = lax.broadcasted_iota(jnp.int32, (NPAD, BGRAPH), 1)
    m = (batch_ref[...] == cols).astype(F32)
    g = lax.dot_general(m, h, (((0,), (0,)), ((), ())),
                        preferred_element_type=F32)
    gm = jnp.mean(g, axis=0, keepdims=True)
    gv = jnp.mean(g * g, axis=0, keepdims=True) - gm * gm
    g = (g - gm) * lax.rsqrt(gv + 1e-5) * fg_ref[...] + fb_ref[...]
    g = jnp.maximum(jnp.dot(g, wl_ref[...], preferred_element_type=F32)
                    + bl_ref[...], 0.0)
    gm = jnp.mean(g, axis=0, keepdims=True)
    gv = jnp.mean(g * g, axis=0, keepdims=True) - gm * gm
    g = (g - gm) * lax.rsqrt(gv + 1e-5) * hg_ref[...] + hb_ref[...]
    g = jnp.dot(g, wc_ref[...], preferred_element_type=F32) + bc_ref[...]
    zmax = jnp.max(g, axis=1, keepdims=True)
    z = g - zmax
    out_ref[...] = z - jnp.log(jnp.sum(jnp.exp(z), axis=1, keepdims=True))


def _tc_final(gu, sqp, t0p, hp, r, we, be, batch2d, fg, fb, wl, bl, hg, hbv,
              wc, bc):
    return pl.pallas_call(
        _tc_final_body,
        compiler_params=_TC_BIG,
        out_shape=jax.ShapeDtypeStruct((BGRAPH, NCLS), F32),
    )(gu, sqp, t0p, hp, r, we, be, batch2d, fg, fb, wl, bl, hg, hbv, wc, bc)


def kernel(x, edge_index, edge_attr, batch, bn_feat_g, bn_feat_b, Wn0, b_n0,
           We0, b_e0, bns_conv_g, bns_conv_b, Wn, b_n, We, b_e, bn_fc_g,
           bn_fc_b, W_lin, b_lin, bn_hid_g, bn_hid_b, W_cls, b_cls):
    row = lambda v: v.reshape(1, -1).astype(F32)
    x_pad = jnp.zeros((NPAD, H), F32).at[:N].set(x)
    src_p = jnp.full((EP,), N, jnp.int32).at[:E].set(edge_index[0])
    dst_p = jnp.full((EP,), N, jnp.int32).at[:E].set(edge_index[1])
    ea_pad = jnp.zeros((EP, DE), F32).at[:E].set(edge_attr)
    batch2d = jnp.full((NPAD, 1), BGRAPH, jnp.int32).at[:N, 0].set(batch)

    zeros_n = jnp.zeros((NPAD,), F32)
    zeros_u = jnp.zeros((NPAD, H), F32)

    degp = _sc_degree(dst_p, zeros_n)
    h1, r = _tc_prep(x_pad, row(bn_feat_g), row(bn_feat_b), Wn0,
                     row(b_n0), degp)
    w, t0p = _sc_wgather(r.reshape(NPAD), src_p, dst_p, zeros_n)
    qp = _tc_edge(ea_pad, We0, row(b_e0), w.reshape(EP, 1))
    sqp = _sc_scatter_q(qp, dst_p, zeros_u)

    hp, u = _tc_c1(h1, r, row(bns_conv_g[0]), row(bns_conv_b[0]),
                   Wn[0], row(b_n[0]))
    for i in range(NLAYER - 1):
        gu = _sc_scatter_u(u, src_p, dst_p, zeros_u)
        hp, u = _tc_mid(gu, sqp, t0p, hp, r, We[i], row(b_e[i]),
                        row(bns_conv_g[i + 1]), row(bns_conv_b[i + 1]),
                        Wn[i + 1], row(b_n[i + 1]))
    gu = _sc_scatter_u(u, src_p, dst_p, zeros_u)
    out = _tc_final(gu, sqp, t0p, hp, r, We[NLAYER - 1], row(b_e[NLAYER - 1]),
                    batch2d, row(bn_fc_g), row(bn_fc_b), W_lin, row(b_lin),
                    row(bn_hid_g), row(bn_hid_b), W_cls, row(b_cls))
    return out
